# baseline (speedup 1.0000x reference)
import functools

import jax
import jax.numpy as jnp
from jax import lax
from jax.experimental import pallas as pl
from jax.experimental.pallas import tpu as pltpu
from jax.experimental.pallas import tpu_sc as plsc

TOKENS = 32768
D = 1024
B = 16
LANES = 16
CHUNKS = D // LANES


def _pool_body(hs_hbm, lens_hbm, out_hbm, lens_v, idx_v, rows_v, sem):
    c = lax.axis_index("c")
    s = lax.axis_index("s")

    @pl.when(jnp.logical_and(c == 0, s == 0))
    def _():
        pltpu.sync_copy(lens_hbm, lens_v)
        lens = lens_v[...]
        lane = lax.broadcasted_iota(jnp.int32, (LANES,), 0)
        cs = jnp.zeros((LANES,), jnp.int32)
        for i in range(B):
            cs = cs + jnp.where(lane >= i, lens[i], jnp.int32(0))
        idx = cs - 1
        idx_v[...] = jnp.where(idx < 0, idx + TOKENS, idx)
        pltpu.async_copy(hs_hbm.at[idx_v], rows_v, sem).wait()

        for r in range(B):
            def sumsq(j, acc):
                v = rows_v[r, pl.ds(j * LANES, LANES)]
                return acc + v * v

            acc = lax.fori_loop(0, CHUNKS, sumsq, jnp.zeros((LANES,), jnp.float32))
            t = acc[0]
            for l in range(1, LANES):
                t = t + acc[l]
            t = jnp.maximum(t, jnp.float32(1e-24))
            bits = lax.bitcast_convert_type(t, jnp.int32)
            ys = lax.bitcast_convert_type(jnp.int32(0x5F3759DF) - (bits >> 1), jnp.float32)
            for _unused in range(3):
                ys = ys * (jnp.float32(1.5) - jnp.float32(0.5) * t * ys * ys)
            y = jnp.full((LANES,), ys, jnp.float32)

            def scale(j, _):
                sl = pl.ds(j * LANES, LANES)
                rows_v[r, sl] = rows_v[r, sl] * y
                return 0

            lax.fori_loop(0, CHUNKS, scale, 0)
        pltpu.sync_copy(rows_v, out_hbm)


def kernel(hidden_states, prompt_lens):
    mesh = plsc.VectorSubcoreMesh(core_axis_name="c", subcore_axis_name="s")
    fn = functools.partial(
        pl.kernel,
        out_type=jax.ShapeDtypeStruct((B, D), jnp.float32),
        mesh=mesh,
        scratch_types=[
            pltpu.VMEM((B,), jnp.int32),
            pltpu.VMEM((B,), jnp.int32),
            pltpu.VMEM((B, D), jnp.float32),
            pltpu.SemaphoreType.DMA,
        ],
    )(_pool_body)
    return fn(hidden_states, prompt_lens)
